# baseline (speedup 1.0000x reference)
import functools

import jax
import jax.numpy as jnp
from jax.experimental import pallas as pl
from jax.experimental.pallas import tpu as pltpu


def _point_sample(feat, points):
    B, C, H, W = feat.shape
    x = points[..., 0] * W - 0.5
    y = points[..., 1] * H - 0.5
    x0 = jnp.floor(x)
    y0 = jnp.floor(y)
    bidx = jnp.arange(B)[:, None]
    out = 0.0
    for dy in (0, 1):
        for dx in (0, 1):
            ix = x0 + dx
            iy = y0 + dy
            wx = 1.0 - jnp.abs(x - ix)
            wy = 1.0 - jnp.abs(y - iy)
            valid = (ix >= 0) & (ix <= W - 1) & (iy >= 0) & (iy <= H - 1)
            ixc = jnp.clip(ix, 0, W - 1).astype(jnp.int32)
            iyc = jnp.clip(iy, 0, H - 1).astype(jnp.int32)
            v = feat[bidx, :, iyc, ixc]
            w = (wx * wy * valid.astype(feat.dtype))[..., None]
            out = out + v * w
    return jnp.transpose(out, (0, 2, 1))


def _sampling_points(mask, N, k, beta, key):
    B, C, H, W = mask.shape
    k1, k2 = jax.random.split(key)
    over = jax.random.uniform(k1, (B, k, 2), dtype=mask.dtype)
    vals = _point_sample(mask, over)
    top2, _ = jax.lax.top_k(jnp.transpose(vals, (0, 2, 1)), 2)
    uncertainty = -1.0 * (top2[..., 0] - top2[..., 1])
    n_imp = int(beta * N)
    _, idx = jax.lax.top_k(uncertainty, n_imp)
    importance = jnp.take_along_axis(over, idx[..., None], axis=1)
    coverage = jax.random.uniform(k2, (B, N - n_imp, 2), dtype=mask.dtype)
    return jnp.concatenate([importance, coverage], axis=1)


def _upx2(x):
    B, C, H, W = x.shape
    return jax.image.resize(x, (B, C, 2 * H, 2 * W), method='bilinear')


def _corner_weight(x, y, x0, y0, dx, dy, W, H):
    ix = x0 + dx
    iy = y0 + dy
    wx = 1.0 - jnp.abs(x - ix)
    wy = 1.0 - jnp.abs(y - iy)
    valid = (ix >= 0) & (ix <= W - 1) & (iy >= 0) & (iy <= H - 1)
    return wx * wy * valid.astype(jnp.float32)


def _mlp_body(W, H, Wt, Ht,
              px_ref, py_ref, frows_ref, trows_ref,
              w1f_ref, w1t_ref, b1_ref, w2_ref, b2_ref,
              w3_ref, b3_ref, w4_ref, b4_ref, out_ref):
    x = px_ref[...]
    y = py_ref[...]

    xf = x * W - 0.5
    yf = y * H - 0.5
    x0 = jnp.floor(xf)
    y0 = jnp.floor(yf)
    featF = jnp.zeros_like(frows_ref[0])
    for i, (dy, dx) in enumerate(((0, 0), (0, 1), (1, 0), (1, 1))):
        w = _corner_weight(xf, yf, x0, y0, dx, dy, W, H)
        featF = featF + w * frows_ref[i]

    xt = x * Wt - 0.5
    yt = y * Ht - 0.5
    xt0 = jnp.floor(xt)
    yt0 = jnp.floor(yt)
    featT = jnp.zeros_like(trows_ref[0])
    for i, (dy, dx) in enumerate(((0, 0), (0, 1), (1, 0), (1, 1))):
        w = _corner_weight(xt, yt, xt0, yt0, dx, dy, Wt, Ht)
        featT = featT + w * trows_ref[i]

    h = jnp.dot(featT, w1t_ref[...], preferred_element_type=jnp.float32)
    h = h + jnp.dot(featF, w1f_ref[...], preferred_element_type=jnp.float32)
    h = jnp.maximum(h + b1_ref[...], 0.0)
    h = jnp.maximum(jnp.dot(h, w2_ref[...], preferred_element_type=jnp.float32)
                    + b2_ref[...], 0.0)
    h = jnp.maximum(jnp.dot(h, w3_ref[...], preferred_element_type=jnp.float32)
                    + b3_ref[...], 0.0)
    out_ref[...] = (jnp.dot(h, w4_ref[...], preferred_element_type=jnp.float32)
                    + b4_ref[...])


def _mlp_stage(px, py, frows, trows, params, W, H, Wt, Ht):
    BN = px.shape[0]
    C = frows.shape[2]
    (w1, b1), (w2, b2), (w3, b3), (w4, b4) = params
    w1t = jnp.pad(jnp.transpose(w1[:, :3]), ((0, 5), (0, 0)))
    w1f = jnp.transpose(w1[:, 3:])
    w4p = jnp.pad(jnp.transpose(w4), ((0, 0), (0, 5)))
    b4p = jnp.pad(b4, (0, 5))
    body = functools.partial(_mlp_body, float(W), float(H), float(Wt), float(Ht))
    return pl.pallas_call(
        body,
        out_shape=jax.ShapeDtypeStruct((BN, 8), jnp.float32),
    )(px, py, frows, trows,
      w1f, w1t, b1[None, :], jnp.transpose(w2), b2[None, :],
      jnp.transpose(w3), b3[None, :], w4p, b4p[None, :])


def _corner_rows(rows, px, py, W, H, HW, N):
    BN = px.shape[0]
    bofs = (jnp.arange(BN, dtype=jnp.int32) // N) * HW
    x = px * W - 0.5
    y = py * H - 0.5
    x0 = jnp.floor(x)
    y0 = jnp.floor(y)
    out = []
    for dy, dx in ((0, 0), (0, 1), (1, 0), (1, 1)):
        ix = jnp.clip(x0 + dx, 0, W - 1).astype(jnp.int32)
        iy = jnp.clip(y0 + dy, 0, H - 1).astype(jnp.int32)
        out.append(jnp.take(rows, bofs + iy * W + ix, axis=0))
    return jnp.stack(out)


def _stage(points, F, T, params):
    B, N, _ = points.shape
    _, C, H, W = F.shape
    _, _, Ht, Wt = T.shape
    BN = B * N
    px = points[..., 0].reshape(BN)
    py = points[..., 1].reshape(BN)
    f_rows = jnp.transpose(F, (0, 2, 3, 1)).reshape(B * H * W, C)
    t_rows = jnp.pad(jnp.transpose(T, (0, 2, 3, 1)).reshape(B * Ht * Wt, 3),
                     ((0, 0), (0, 5)))
    frows = _corner_rows(f_rows, px, py, W, H, H * W, N)
    trows = _corner_rows(t_rows, px, py, Wt, Ht, Ht * Wt, N)
    out = _mlp_stage(px[:, None], py[:, None], frows, trows, params, W, H, Wt, Ht)
    return jnp.transpose(out[:, :3].reshape(B, N, 3), (0, 2, 1))


def kernel(x1, x2, x3, coarse, mlp3, mlp2, mlp1):
    key = jax.random.key(42)
    k3, k4, k5 = jax.random.split(key, 3)
    temp3 = _upx2(coarse)
    points3 = _sampling_points(jax.nn.softmax(temp3, axis=1), N=200, k=800,
                               beta=0.95, key=k3)
    rend3 = _stage(points3, x3, temp3, mlp3)
    temp4 = _upx2(temp3)
    points4 = _sampling_points(jax.nn.softmax(temp4, axis=1), N=400, k=800,
                               beta=0.95, key=k4)
    rend4 = _stage(points4, x2, temp4, mlp2)
    temp5 = _upx2(temp4)
    points5 = _sampling_points(jax.nn.softmax(temp5, axis=1), N=800, k=800,
                               beta=0.95, key=k5)
    rend5 = _stage(points5, x1, temp5, mlp1)
    return (points3, rend3, points4, rend4, points5, rend5)


# speedup vs baseline: 1.1227x; 1.1227x over previous
import functools

import jax
import jax.numpy as jnp
from jax.experimental import pallas as pl
from jax.experimental.pallas import tpu as pltpu


def _point_sample(feat, points):
    B, C, H, W = feat.shape
    x = points[..., 0] * W - 0.5
    y = points[..., 1] * H - 0.5
    x0 = jnp.floor(x)
    y0 = jnp.floor(y)
    bidx = jnp.arange(B)[:, None]
    out = 0.0
    for dy in (0, 1):
        for dx in (0, 1):
            ix = x0 + dx
            iy = y0 + dy
            wx = 1.0 - jnp.abs(x - ix)
            wy = 1.0 - jnp.abs(y - iy)
            valid = (ix >= 0) & (ix <= W - 1) & (iy >= 0) & (iy <= H - 1)
            ixc = jnp.clip(ix, 0, W - 1).astype(jnp.int32)
            iyc = jnp.clip(iy, 0, H - 1).astype(jnp.int32)
            v = feat[bidx, :, iyc, ixc]
            w = (wx * wy * valid.astype(feat.dtype))[..., None]
            out = out + v * w
    return jnp.transpose(out, (0, 2, 1))


def _sampling_points(mask, N, k, beta, key):
    B, C, H, W = mask.shape
    k1, k2 = jax.random.split(key)
    over = jax.random.uniform(k1, (B, k, 2), dtype=mask.dtype)
    vals = _point_sample(mask, over)
    top2, _ = jax.lax.top_k(jnp.transpose(vals, (0, 2, 1)), 2)
    uncertainty = -1.0 * (top2[..., 0] - top2[..., 1])
    n_imp = int(beta * N)
    _, idx = jax.lax.top_k(uncertainty, n_imp)
    importance = jnp.take_along_axis(over, idx[..., None], axis=1)
    coverage = jax.random.uniform(k2, (B, N - n_imp, 2), dtype=mask.dtype)
    return jnp.concatenate([importance, coverage], axis=1)


def _upx2(x):
    B, C, H, W = x.shape
    return jax.image.resize(x, (B, C, 2 * H, 2 * W), method='bilinear')


def _corner_weight(x, y, x0, y0, dx, dy, W, H):
    ix = x0 + dx
    iy = y0 + dy
    wx = 1.0 - jnp.abs(x - ix)
    wy = 1.0 - jnp.abs(y - iy)
    valid = (ix >= 0) & (ix <= W - 1) & (iy >= 0) & (iy <= H - 1)
    return wx * wy * valid.astype(jnp.float32)


def _mlp_body(W, H,
              px_ref, py_ref, frows_ref, featT_ref,
              w1f_ref, w1t_ref, b1_ref, w2_ref, b2_ref,
              w3_ref, b3_ref, w4_ref, b4_ref, out_ref):
    x = px_ref[...]
    y = py_ref[...]

    xf = x * W - 0.5
    yf = y * H - 0.5
    x0 = jnp.floor(xf)
    y0 = jnp.floor(yf)
    featF = jnp.zeros_like(frows_ref[0])
    for i, (dy, dx) in enumerate(((0, 0), (0, 1), (1, 0), (1, 1))):
        w = _corner_weight(xf, yf, x0, y0, dx, dy, W, H)
        featF = featF + w * frows_ref[i]

    h = jnp.dot(featT_ref[...], w1t_ref[...], preferred_element_type=jnp.float32)
    h = h + jnp.dot(featF, w1f_ref[...], preferred_element_type=jnp.float32)
    h = jnp.maximum(h + b1_ref[...], 0.0)
    h = jnp.maximum(jnp.dot(h, w2_ref[...], preferred_element_type=jnp.float32)
                    + b2_ref[...], 0.0)
    h = jnp.maximum(jnp.dot(h, w3_ref[...], preferred_element_type=jnp.float32)
                    + b3_ref[...], 0.0)
    out_ref[...] = (jnp.dot(h, w4_ref[...], preferred_element_type=jnp.float32)
                    + b4_ref[...])


def _mlp_stage(px, py, frows, featT, params, W, H):
    BN = px.shape[0]
    (w1, b1), (w2, b2), (w3, b3), (w4, b4) = params
    w1t = jnp.pad(jnp.transpose(w1[:, :3]), ((0, 5), (0, 0)))
    w1f = jnp.transpose(w1[:, 3:])
    w4p = jnp.pad(jnp.transpose(w4), ((0, 0), (0, 5)))
    b4p = jnp.pad(b4, (0, 5))
    body = functools.partial(_mlp_body, float(W), float(H))
    return pl.pallas_call(
        body,
        out_shape=jax.ShapeDtypeStruct((BN, 8), jnp.float32),
    )(px, py, frows, featT,
      w1f, w1t, b1[None, :], jnp.transpose(w2), b2[None, :],
      jnp.transpose(w3), b3[None, :], w4p, b4p[None, :])


def _corner_rows(rows, px, py, W, H, HW, N):
    BN = px.shape[0]
    bofs = (jnp.arange(BN, dtype=jnp.int32) // N) * HW
    x = px * W - 0.5
    y = py * H - 0.5
    x0 = jnp.floor(x)
    y0 = jnp.floor(y)
    out = []
    for dy, dx in ((0, 0), (0, 1), (1, 0), (1, 1)):
        ix = jnp.clip(x0 + dx, 0, W - 1).astype(jnp.int32)
        iy = jnp.clip(y0 + dy, 0, H - 1).astype(jnp.int32)
        out.append(jnp.take(rows, bofs + iy * W + ix, axis=0))
    return jnp.stack(out)


def _stage(points, F, T, params):
    B, N, _ = points.shape
    _, C, H, W = F.shape
    _, _, Ht, Wt = T.shape
    BN = B * N
    px = points[..., 0].reshape(BN)
    py = points[..., 1].reshape(BN)
    f_rows = jnp.transpose(F, (0, 2, 3, 1)).reshape(B * H * W, C)
    frows = _corner_rows(f_rows, px, py, W, H, H * W, N)
    featT = jnp.pad(jnp.transpose(_point_sample(T, points), (0, 2, 1))
                    .reshape(BN, 3), ((0, 0), (0, 5)))
    out = _mlp_stage(px[:, None], py[:, None], frows, featT, params, W, H)
    return jnp.transpose(out[:, :3].reshape(B, N, 3), (0, 2, 1))


def kernel(x1, x2, x3, coarse, mlp3, mlp2, mlp1):
    key = jax.random.key(42)
    k3, k4, k5 = jax.random.split(key, 3)
    temp3 = _upx2(coarse)
    points3 = _sampling_points(jax.nn.softmax(temp3, axis=1), N=200, k=800,
                               beta=0.95, key=k3)
    rend3 = _stage(points3, x3, temp3, mlp3)
    temp4 = _upx2(temp3)
    points4 = _sampling_points(jax.nn.softmax(temp4, axis=1), N=400, k=800,
                               beta=0.95, key=k4)
    rend4 = _stage(points4, x2, temp4, mlp2)
    temp5 = _upx2(temp4)
    points5 = _sampling_points(jax.nn.softmax(temp5, axis=1), N=800, k=800,
                               beta=0.95, key=k5)
    rend5 = _stage(points5, x1, temp5, mlp1)
    return (points3, rend3, points4, rend4, points5, rend5)


# speedup vs baseline: 1.4885x; 1.3258x over previous
import functools

import jax
import jax.numpy as jnp
from jax import lax
from jax.experimental import pallas as pl
from jax.experimental.pallas import tpu as pltpu
from jax.experimental.pallas import tpu_sc as plsc


def _point_sample(feat, points):
    B, C, H, W = feat.shape
    x = points[..., 0] * W - 0.5
    y = points[..., 1] * H - 0.5
    x0 = jnp.floor(x)
    y0 = jnp.floor(y)
    bidx = jnp.arange(B)[:, None]
    out = 0.0
    for dy in (0, 1):
        for dx in (0, 1):
            ix = x0 + dx
            iy = y0 + dy
            wx = 1.0 - jnp.abs(x - ix)
            wy = 1.0 - jnp.abs(y - iy)
            valid = (ix >= 0) & (ix <= W - 1) & (iy >= 0) & (iy <= H - 1)
            ixc = jnp.clip(ix, 0, W - 1).astype(jnp.int32)
            iyc = jnp.clip(iy, 0, H - 1).astype(jnp.int32)
            v = feat[bidx, :, iyc, ixc]
            w = (wx * wy * valid.astype(feat.dtype))[..., None]
            out = out + v * w
    return jnp.transpose(out, (0, 2, 1))


def _sampling_points(mask, N, k, beta, key):
    B, C, H, W = mask.shape
    k1, k2 = jax.random.split(key)
    over = jax.random.uniform(k1, (B, k, 2), dtype=mask.dtype)
    vals = _point_sample(mask, over)
    top2, _ = jax.lax.top_k(jnp.transpose(vals, (0, 2, 1)), 2)
    uncertainty = -1.0 * (top2[..., 0] - top2[..., 1])
    n_imp = int(beta * N)
    _, idx = jax.lax.top_k(uncertainty, n_imp)
    importance = jnp.take_along_axis(over, idx[..., None], axis=1)
    coverage = jax.random.uniform(k2, (B, N - n_imp, 2), dtype=mask.dtype)
    return jnp.concatenate([importance, coverage], axis=1)


def _upx2(x):
    B, C, H, W = x.shape
    return jax.image.resize(x, (B, C, 2 * H, 2 * W), method='bilinear')


def _mlp_body(featF_ref, featT_ref,
              w1f_ref, w1t_ref, b1_ref, w2_ref, b2_ref,
              w3_ref, b3_ref, w4_ref, b4_ref, out_ref):
    h = jnp.dot(featT_ref[...], w1t_ref[...], preferred_element_type=jnp.float32)
    h = h + jnp.dot(featF_ref[...], w1f_ref[...], preferred_element_type=jnp.float32)
    h = jnp.maximum(h + b1_ref[...], 0.0)
    h = jnp.maximum(jnp.dot(h, w2_ref[...], preferred_element_type=jnp.float32)
                    + b2_ref[...], 0.0)
    h = jnp.maximum(jnp.dot(h, w3_ref[...], preferred_element_type=jnp.float32)
                    + b3_ref[...], 0.0)
    out_ref[...] = (jnp.dot(h, w4_ref[...], preferred_element_type=jnp.float32)
                    + b4_ref[...])


def _mlp_stage(featF, featT, params):
    BN = featF.shape[0]
    (w1, b1), (w2, b2), (w3, b3), (w4, b4) = params
    w1t = jnp.pad(jnp.transpose(w1[:, :3]), ((0, 5), (0, 0)))
    w1f = jnp.transpose(w1[:, 3:])
    w4p = jnp.pad(jnp.transpose(w4), ((0, 0), (0, 5)))
    b4p = jnp.pad(b4, (0, 5))
    return pl.pallas_call(
        _mlp_body,
        out_shape=jax.ShapeDtypeStruct((BN, 8), jnp.float32),
    )(featF, featT,
      w1f, w1t, b1[None, :], jnp.transpose(w2), b2[None, :],
      jnp.transpose(w3), b3[None, :], w4p, b4p[None, :])


_NWORKERS = 32


def _sc_gather_combine(f_rows, px, py, W, H, HW, N):
    BN = px.shape[0]
    C = f_rows.shape[1]
    nchunks = BN // 16
    kmax = (nchunks + _NWORKERS - 1) // _NWORKERS
    Wf, Hf = float(W), float(H)
    mesh = plsc.VectorSubcoreMesh(core_axis_name="c", subcore_axis_name="s")

    @functools.partial(
        pl.kernel,
        out_type=jax.ShapeDtypeStruct((BN, C), jnp.float32),
        mesh=mesh,
        compiler_params=pltpu.CompilerParams(needs_layout_passes=False),
        scratch_types=[
            pltpu.VMEM((16,), jnp.float32),
            pltpu.VMEM((16,), jnp.float32),
            pltpu.VMEM((16, C), jnp.float32),
            pltpu.VMEM((16, C), jnp.float32),
            pltpu.VMEM((16, C), jnp.float32),
            pltpu.VMEM((16, C), jnp.float32),
            pltpu.VMEM((16, C), jnp.float32),
            pltpu.SemaphoreType.DMA,
        ],
    )
    def k(frows_hbm, px_hbm, py_hbm, out_hbm,
          px_v, py_v, r00, r01, r10, r11, out_v, sem):
        wid = lax.axis_index("s") * 2 + lax.axis_index("c")
        lanes = lax.iota(jnp.int32, 16)
        for kk in range(kmax):
            chunk = wid + kk * _NWORKERS

            @pl.when(chunk < nchunks)
            def _():
                base = chunk * 16
                pltpu.sync_copy(px_hbm.at[pl.ds(base, 16)], px_v)
                pltpu.sync_copy(py_hbm.at[pl.ds(base, 16)], py_v)
                x = px_v[...] * Wf - 0.5
                y = py_v[...] * Hf - 0.5
                xi = x.astype(jnp.int32)
                x0 = xi - jnp.where(x < xi.astype(jnp.float32), 1, 0)
                yi = y.astype(jnp.int32)
                y0 = yi - jnp.where(y < yi.astype(jnp.float32), 1, 0)
                bofs = jnp.where(base + lanes >= N, HW, 0)
                ix0 = jnp.clip(x0, 0, W - 1)
                ix1 = jnp.clip(x0 + 1, 0, W - 1)
                iy0 = jnp.clip(y0, 0, H - 1)
                iy1 = jnp.clip(y0 + 1, 0, H - 1)
                c0 = pltpu.async_copy(frows_hbm.at[bofs + iy0 * W + ix0], r00, sem)
                c1 = pltpu.async_copy(frows_hbm.at[bofs + iy0 * W + ix1], r01, sem)
                c2 = pltpu.async_copy(frows_hbm.at[bofs + iy1 * W + ix0], r10, sem)
                c3 = pltpu.async_copy(frows_hbm.at[bofs + iy1 * W + ix1], r11, sem)
                x0f = x0.astype(jnp.float32)
                y0f = y0.astype(jnp.float32)
                wx0 = 1.0 - jnp.abs(x - x0f)
                wx1 = 1.0 - jnp.abs(x - (x0f + 1.0))
                wy0 = 1.0 - jnp.abs(y - y0f)
                wy1 = 1.0 - jnp.abs(y - (y0f + 1.0))
                vx0 = ((x0 >= 0) & (x0 <= W - 1)).astype(jnp.float32)
                vx1 = ((x0 + 1 >= 0) & (x0 + 1 <= W - 1)).astype(jnp.float32)
                vy0 = ((y0 >= 0) & (y0 <= H - 1)).astype(jnp.float32)
                vy1 = ((y0 + 1 >= 0) & (y0 + 1 <= H - 1)).astype(jnp.float32)
                w00 = wx0 * wy0 * (vx0 * vy0)
                w01 = wx1 * wy0 * (vx1 * vy0)
                w10 = wx0 * wy1 * (vx0 * vy1)
                w11 = wx1 * wy1 * (vx1 * vy1)
                c0.wait()
                c1.wait()
                c2.wait()
                c3.wait()

                def body(ci, carry):
                    idxc = jnp.zeros((16,), jnp.int32) + ci
                    v = (plsc.load_gather(r00, [lanes, idxc]) * w00
                         + plsc.load_gather(r01, [lanes, idxc]) * w01
                         + plsc.load_gather(r10, [lanes, idxc]) * w10
                         + plsc.load_gather(r11, [lanes, idxc]) * w11)
                    plsc.store_scatter(out_v, [lanes, idxc], v)
                    return carry

                lax.fori_loop(0, C, body, 0)
                pltpu.sync_copy(out_v, out_hbm.at[pl.ds(base, 16)])

    return k(f_rows, px, py)


def _stage(points, F, T, params):
    B, N, _ = points.shape
    _, C, H, W = F.shape
    BN = B * N
    px = points[..., 0].reshape(BN)
    py = points[..., 1].reshape(BN)
    f_rows = jnp.transpose(F, (0, 2, 3, 1)).reshape(B * H * W, C)
    featF = _sc_gather_combine(f_rows, px, py, W, H, H * W, N)
    featT = jnp.pad(jnp.transpose(_point_sample(T, points), (0, 2, 1))
                    .reshape(BN, 3), ((0, 0), (0, 5)))
    out = _mlp_stage(featF, featT, params)
    return jnp.transpose(out[:, :3].reshape(B, N, 3), (0, 2, 1))


def kernel(x1, x2, x3, coarse, mlp3, mlp2, mlp1):
    key = jax.random.key(42)
    k3, k4, k5 = jax.random.split(key, 3)
    temp3 = _upx2(coarse)
    points3 = _sampling_points(jax.nn.softmax(temp3, axis=1), N=200, k=800,
                               beta=0.95, key=k3)
    rend3 = _stage(points3, x3, temp3, mlp3)
    temp4 = _upx2(temp3)
    points4 = _sampling_points(jax.nn.softmax(temp4, axis=1), N=400, k=800,
                               beta=0.95, key=k4)
    rend4 = _stage(points4, x2, temp4, mlp2)
    temp5 = _upx2(temp4)
    points5 = _sampling_points(jax.nn.softmax(temp5, axis=1), N=800, k=800,
                               beta=0.95, key=k5)
    rend5 = _stage(points5, x1, temp5, mlp1)
    return (points3, rend3, points4, rend4, points5, rend5)
